# speedup vs baseline: 1.2733x; 1.2733x over previous
import functools

import jax
import jax.numpy as jnp
from jax import lax
from jax.experimental import pallas as pl
from jax.experimental.pallas import tpu as pltpu
from jax.experimental.pallas import tpu_sc as plsc

HIDDEN = 768
EPS = 1e-12

NC = 2
NS = 16
NW = NC * NS

TOKENS = 8192
B_PER_W = TOKENS // NW
CHUNK = 64
N_CHUNKS = B_PER_W // CHUNK

TOK_BLK = 512


def _sc_gather(table, ids):
    mesh = plsc.VectorSubcoreMesh(core_axis_name="c", subcore_axis_name="s")

    @functools.partial(
        pl.kernel,
        mesh=mesh,
        out_type=jax.ShapeDtypeStruct((TOKENS, HIDDEN), jnp.float32),
        scratch_types=[
            pltpu.VMEM((B_PER_W,), jnp.int32),
            pltpu.VMEM((CHUNK, HIDDEN), jnp.float32),
            pltpu.SemaphoreType.DMA,
        ],
    )
    def k(table_hbm, idx_hbm, out_hbm, idx_v, rows_v, sem):
        wid = lax.axis_index("s") * NC + lax.axis_index("c")
        base = wid * B_PER_W
        pltpu.sync_copy(idx_hbm.at[pl.ds(base, B_PER_W)], idx_v)

        @pl.loop(0, N_CHUNKS)
        def _(c):
            off = c * CHUNK
            pltpu.async_copy(
                table_hbm.at[idx_v.at[pl.ds(off, CHUNK)]], rows_v, sem
            ).wait()
            pltpu.sync_copy(rows_v, out_hbm.at[pl.ds(base + off, CHUNK)])

    return k(table, ids)


def _ln_body(x_ref, pos_ref, g_ref, b_ref, o_ref):
    x = x_ref[...] + pos_ref[...]
    mean = jnp.mean(x, axis=1, keepdims=True)
    xc = x - mean
    var = jnp.mean(xc * xc, axis=1, keepdims=True)
    inv = lax.rsqrt(var + EPS)
    o_ref[...] = xc * inv * g_ref[...] + b_ref[...]


def _tc_ln(gathered, pos, gamma, beta, seq_len):
    blocks_per_seq = seq_len // TOK_BLK
    return pl.pallas_call(
        _ln_body,
        grid=(TOKENS // TOK_BLK,),
        in_specs=[
            pl.BlockSpec((TOK_BLK, HIDDEN), lambda i: (i, 0)),
            pl.BlockSpec((TOK_BLK, HIDDEN), lambda i: (i % blocks_per_seq, 0)),
            pl.BlockSpec((1, HIDDEN), lambda i: (0, 0)),
            pl.BlockSpec((1, HIDDEN), lambda i: (0, 0)),
        ],
        out_specs=pl.BlockSpec((TOK_BLK, HIDDEN), lambda i: (i, 0)),
        out_shape=jax.ShapeDtypeStruct((TOKENS, HIDDEN), jnp.float32),
    )(gathered, pos, gamma.reshape(1, HIDDEN), beta.reshape(1, HIDDEN))


def kernel(input_ids, word_embeddings, position_embeddings, ln_gamma, ln_beta):
    batch, seq = input_ids.shape
    assert batch * seq == TOKENS
    ids = input_ids.reshape(-1).astype(jnp.int32)
    gathered = _sc_gather(word_embeddings, ids)
    out = _tc_ln(gathered, position_embeddings, ln_gamma, ln_beta, seq)
    return out.reshape(batch, seq, HIDDEN)


# speedup vs baseline: 1.3440x; 1.0555x over previous
import functools

import jax
import jax.numpy as jnp
from jax import lax
from jax.experimental import pallas as pl
from jax.experimental.pallas import tpu as pltpu
from jax.experimental.pallas import tpu_sc as plsc

HIDDEN = 768
EPS = 1e-12

NC = 2
NS = 16
NW = NC * NS

TOKENS = 8192
B_PER_W = TOKENS // NW
CHUNK = 64
N_CHUNKS = B_PER_W // CHUNK

TOK_BLK = 512


def _sc_gather(table, ids):
    mesh = plsc.VectorSubcoreMesh(core_axis_name="c", subcore_axis_name="s")

    @functools.partial(
        pl.kernel,
        mesh=mesh,
        out_type=jax.ShapeDtypeStruct((TOKENS, HIDDEN), jnp.float32),
        scratch_types=[
            pltpu.VMEM((B_PER_W,), jnp.int32),
            pltpu.VMEM((CHUNK, HIDDEN), jnp.float32),
            pltpu.VMEM((CHUNK, HIDDEN), jnp.float32),
            pltpu.SemaphoreType.DMA,
            pltpu.SemaphoreType.DMA,
            pltpu.SemaphoreType.DMA,
            pltpu.SemaphoreType.DMA,
        ],
    )
    def k(table_hbm, idx_hbm, out_hbm, idx_v, rows_a, rows_b, g0, g1, s0, s1):
        wid = lax.axis_index("s") * NC + lax.axis_index("c")
        base = wid * B_PER_W
        pltpu.sync_copy(idx_hbm.at[pl.ds(base, B_PER_W)], idx_v)

        bufs = (rows_a, rows_b)
        gsems = (g0, g1)
        ssems = (s0, s1)
        gathers = [None, None]
        stores = [None, None]
        for c in range(N_CHUNKS):
            p = c % 2
            if stores[p] is not None:
                stores[p].wait()
            gathers[p] = pltpu.async_copy(
                table_hbm.at[idx_v.at[pl.ds(c * CHUNK, CHUNK)]], bufs[p], gsems[p]
            )
            if c >= 1:
                q = (c - 1) % 2
                gathers[q].wait()
                stores[q] = pltpu.async_copy(
                    bufs[q], out_hbm.at[pl.ds(base + (c - 1) * CHUNK, CHUNK)], ssems[q]
                )
        last = (N_CHUNKS - 1) % 2
        gathers[last].wait()
        stores[last] = pltpu.async_copy(
            bufs[last], out_hbm.at[pl.ds(base + (N_CHUNKS - 1) * CHUNK, CHUNK)],
            ssems[last],
        )
        stores[1 - last].wait()
        stores[last].wait()

    return k(table, ids)


def _ln_body(x_ref, pos_ref, g_ref, b_ref, o_ref):
    x = x_ref[...] + pos_ref[...]
    mean = jnp.mean(x, axis=1, keepdims=True)
    xc = x - mean
    var = jnp.mean(xc * xc, axis=1, keepdims=True)
    inv = lax.rsqrt(var + EPS)
    o_ref[...] = xc * inv * g_ref[...] + b_ref[...]


def _tc_ln(gathered, pos, gamma, beta, batch, seq_len):
    bps = seq_len // TOK_BLK
    return pl.pallas_call(
        _ln_body,
        grid=(bps, batch),
        in_specs=[
            pl.BlockSpec((TOK_BLK, HIDDEN), lambda i, j: (j * bps + i, 0)),
            pl.BlockSpec((TOK_BLK, HIDDEN), lambda i, j: (i, 0)),
            pl.BlockSpec((1, HIDDEN), lambda i, j: (0, 0)),
            pl.BlockSpec((1, HIDDEN), lambda i, j: (0, 0)),
        ],
        out_specs=pl.BlockSpec((TOK_BLK, HIDDEN), lambda i, j: (j * bps + i, 0)),
        out_shape=jax.ShapeDtypeStruct((TOKENS, HIDDEN), jnp.float32),
    )(gathered, pos, gamma.reshape(1, HIDDEN), beta.reshape(1, HIDDEN))


def kernel(input_ids, word_embeddings, position_embeddings, ln_gamma, ln_beta):
    batch, seq = input_ids.shape
    assert batch * seq == TOKENS
    ids = input_ids.reshape(-1).astype(jnp.int32)
    gathered = _sc_gather(word_embeddings, ids)
    out = _tc_ln(gathered, position_embeddings, ln_gamma, ln_beta, batch, seq)
    return out.reshape(batch, seq, HIDDEN)
